# baseline (speedup 1.0000x reference)
import functools

import jax
import jax.numpy as jnp
from jax import lax
from jax.experimental import pallas as pl
from jax.experimental.pallas import tpu as pltpu
from jax.experimental.pallas import tpu_sc as plsc

_B = 16384
_NS = 26
_ND = 13
_V = 1000000
_NC = 2
_NSUB = 16
_NW = _NC * _NSUB
_BPW = _B // _NW
_L = 16
_CHUNK = 128
_NCH = _NS * _BPW // _CHUNK
_QPF = _BPW // _CHUNK
_K = 8


def _sc_body(xs_hbm, xd_hbm, emb_hbm, dwb_hbm, out_hbm,
             xs_v, idx_v, vals_v, xd_v, dw_v, acc_v, sem):
    w = lax.axis_index("s") * _NC + lax.axis_index("c")
    base = w * _BPW

    pltpu.sync_copy(xs_hbm.at[w], xs_v)
    pltpu.sync_copy(xd_hbm.at[w], xd_v)
    pltpu.sync_copy(dwb_hbm, dw_v)

    def build(j, carry):
        off = (j // _QPF) * _V
        for i in range(_CHUNK // _L):
            ids = xs_v[j, pl.ds(i * _L, _L)]
            idx_v[j, pl.ds(i * _L, _L)] = ids.astype(jnp.int32) + off
        return carry

    lax.fori_loop(0, _NCH, build, 0)

    def gather(g, carry):
        handles = []
        for b in range(_K):
            j = g * _K + b
            handles.append(
                pltpu.async_copy(emb_hbm.at[idx_v.at[j]], vals_v.at[j], sem))
        for h in handles:
            h.wait()
        return carry

    lax.fori_loop(0, _NCH // _K, gather, 0)

    def reduce(c, carry):
        q = c // (_CHUNK // _L)
        off = (c % (_CHUNK // _L)) * _L
        acc = vals_v[q, pl.ds(off, _L)]
        for f in range(1, _NS):
            acc = acc + vals_v[f * _QPF + q, pl.ds(off, _L)]
        for k in range(_ND):
            acc = acc + xd_v[k, pl.ds(c * _L, _L)] * dw_v[k]
        acc_v[pl.ds(c * _L, _L)] = acc
        return carry

    lax.fori_loop(0, _BPW // _L, reduce, 0)

    pltpu.sync_copy(acc_v, out_hbm.at[pl.ds(base, _BPW)])


@jax.jit
def kernel(X, emb_tables, dense_weight):
    xs_r = (X[:, :_NS].T.reshape(_NS, _NW, _BPW)
            .transpose(1, 0, 2).reshape(_NW, _NCH, _CHUNK))
    xd_r = X[:, _NS:].T.reshape(_ND, _NW, _BPW).transpose(1, 0, 2)
    emb_flat = emb_tables.reshape(_NS * _V)
    dwb = jnp.broadcast_to(dense_weight.reshape(_ND, 1), (_ND, _L))
    run = pl.kernel(
        _sc_body,
        out_type=jax.ShapeDtypeStruct((_B,), jnp.float32),
        mesh=plsc.VectorSubcoreMesh(core_axis_name="c", subcore_axis_name="s"),
        scratch_types=[
            pltpu.VMEM((_NCH, _CHUNK), jnp.float32),
            pltpu.VMEM((_NCH, _CHUNK), jnp.int32),
            pltpu.VMEM((_NCH, _CHUNK), jnp.float32),
            pltpu.VMEM((_ND, _BPW), jnp.float32),
            pltpu.VMEM((_ND, _L), jnp.float32),
            pltpu.VMEM((_BPW,), jnp.float32),
            pltpu.SemaphoreType.DMA,
        ],
    )
    out = run(xs_r, xd_r, emb_flat, dwb)
    return out.reshape(_B, 1)


# speedup vs baseline: 1.0004x; 1.0004x over previous
import jax
import jax.numpy as jnp
from jax import lax
from jax.experimental import pallas as pl
from jax.experimental.pallas import tpu as pltpu
from jax.experimental.pallas import tpu_sc as plsc

_B = 16384
_NS = 26
_ND = 13
_NF = _NS + _ND
_V = 1000000
_NC = 2
_NSUB = 16
_NW = _NC * _NSUB
_BPW = _B // _NW
_L = 16
_CHUNK = 128
_NCH = _NS * _BPW // _CHUNK
_QPF = _BPW // _CHUNK
_K = 8


def _sc_body(x_hbm, emb_hbm, dwb_hbm, out_hbm,
             cols_v, idx_v, vals_v, dw_v, acc_v, sem):
    w = lax.axis_index("s") * _NC + lax.axis_index("c")
    base = w * _BPW

    pltpu.sync_copy(dwb_hbm, dw_v)

    pltpu.sync_copy(x_hbm.at[w], cols_v)

    def build(j, carry):
        f = j // _QPF
        q = j % _QPF
        off = f * _V
        for i in range(_CHUNK // _L):
            ids = cols_v[f, pl.ds(q * _CHUNK + i * _L, _L)]
            idx_v[j, pl.ds(i * _L, _L)] = ids.astype(jnp.int32) + off
        return carry

    lax.fori_loop(0, _NCH, build, 0)

    def gather(g, carry):
        handles = []
        for b in range(_K):
            j = g * _K + b
            handles.append(
                pltpu.async_copy(emb_hbm.at[idx_v.at[j]], vals_v.at[j], sem))
        for h in handles:
            h.wait()
        return carry

    lax.fori_loop(0, _NCH // _K, gather, 0)

    def reduce(c, carry):
        q = c // (_CHUNK // _L)
        off = (c % (_CHUNK // _L)) * _L
        acc = vals_v[q, pl.ds(off, _L)]
        for f in range(1, _NS):
            acc = acc + vals_v[f * _QPF + q, pl.ds(off, _L)]
        for k in range(_ND):
            acc = acc + cols_v[_NS + k, pl.ds(c * _L, _L)] * dw_v[k]
        acc_v[pl.ds(c * _L, _L)] = acc
        return carry

    lax.fori_loop(0, _BPW // _L, reduce, 0)

    pltpu.sync_copy(acc_v, out_hbm.at[pl.ds(base, _BPW)])


@jax.jit
def kernel(X, emb_tables, dense_weight):
    x_r = X.reshape(_NW, _BPW, _NF).transpose(0, 2, 1)
    emb_flat = emb_tables.reshape(_NS * _V)
    dwb = jnp.broadcast_to(dense_weight.reshape(_ND, 1), (_ND, _L))
    run = pl.kernel(
        _sc_body,
        out_type=jax.ShapeDtypeStruct((_B,), jnp.float32),
        mesh=plsc.VectorSubcoreMesh(core_axis_name="c", subcore_axis_name="s"),
        compiler_params=pltpu.CompilerParams(use_tc_tiling_on_sc=False),
        scratch_types=[
            pltpu.VMEM((_NF, _BPW), jnp.float32),
            pltpu.VMEM((_NCH, _CHUNK), jnp.int32),
            pltpu.VMEM((_NCH, _CHUNK), jnp.float32),
            pltpu.VMEM((_ND, _L), jnp.float32),
            pltpu.VMEM((_BPW,), jnp.float32),
            pltpu.SemaphoreType.DMA,
        ],
    )
    out = run(x_r, emb_flat, dwb)
    return out.reshape(_B, 1)


# speedup vs baseline: 1.4376x; 1.4370x over previous
import jax
import jax.numpy as jnp
from jax import lax
from jax.experimental import pallas as pl
from jax.experimental.pallas import tpu as pltpu
from jax.experimental.pallas import tpu_sc as plsc

_B = 16384
_NS = 26
_ND = 13
_V = 1000000
_NC = 2
_NSUB = 16
_NW = _NC * _NSUB
_BPW = _B // _NW
_L = 16
_CHUNK = 128
_NCH = _NS * _BPW // _CHUNK
_QPF = _BPW // _CHUNK
_K = 8

_CT = (_V + 127) // 128
_BAND = _CT * 1024
_PHYS = 4 * _BAND


def _sc_body(xs_hbm, xd_hbm, emb_hbm, dwb_hbm, out_hbm,
             xs_v, idx_v, vals_v, xd_v, dw_v, acc_v, sem):
    w = lax.axis_index("s") * _NC + lax.axis_index("c")
    base = w * _BPW

    pltpu.sync_copy(xs_hbm.at[w], xs_v)
    pltpu.sync_copy(xd_hbm.at[w], xd_v)
    pltpu.sync_copy(dwb_hbm, dw_v)

    def build(j, carry):
        off = (j // _QPF) * _V
        for i in range(_CHUNK // _L):
            ids = xs_v[j, pl.ds(i * _L, _L)]
            idx_v[j, pl.ds(i * _L, _L)] = ids.astype(jnp.int32) + off
        return carry

    lax.fori_loop(0, _NCH, build, 0)

    def gather(g, carry):
        handles = []
        for b in range(_K):
            j = g * _K + b
            handles.append(
                pltpu.async_copy(emb_hbm.at[idx_v.at[j]], vals_v.at[j], sem))
        for h in handles:
            h.wait()
        return carry

    lax.fori_loop(0, _NCH // _K, gather, 0)

    def reduce(c, carry):
        q = c // (_CHUNK // _L)
        off = (c % (_CHUNK // _L)) * _L
        acc = vals_v[q, pl.ds(off, _L)]
        for f in range(1, _NS):
            acc = acc + vals_v[f * _QPF + q, pl.ds(off, _L)]
        for k in range(_ND):
            acc = acc + xd_v[k, pl.ds(c * _L, _L)] * dw_v[k]
        acc_v[pl.ds(c * _L, _L)] = acc
        return carry

    lax.fori_loop(0, _BPW // _L, reduce, 0)

    pltpu.sync_copy(acc_v, out_hbm.at[pl.ds(base, _BPW)])


@jax.jit
def kernel(X, emb_tables, dense_weight):
    xs_r = (X[:, :_NS].reshape(_NW, _QPF, _CHUNK, _NS)
            .transpose(0, 3, 1, 2).reshape(_NW, _NCH, _CHUNK))
    xd_r = jnp.concatenate(
        [X[:, _NS:], jnp.zeros((_B, 3), jnp.float32)], axis=1
    ).reshape(_NW, _BPW, _ND + 3).transpose(0, 2, 1)
    dwb = jnp.broadcast_to(dense_weight.reshape(_ND, 1), (_ND, _L))
    emb_flat = jnp.concatenate([emb_tables[f] for f in range(_NS)])
    run = pl.kernel(
        _sc_body,
        out_type=jax.ShapeDtypeStruct((_B,), jnp.float32),
        mesh=plsc.VectorSubcoreMesh(core_axis_name="c", subcore_axis_name="s"),
        scratch_types=[
            pltpu.VMEM((_NCH, _CHUNK), jnp.float32),
            pltpu.VMEM((_NCH, _CHUNK), jnp.int32),
            pltpu.VMEM((_NCH, _CHUNK), jnp.float32),
            pltpu.VMEM((_ND + 3, _BPW), jnp.float32),
            pltpu.VMEM((_ND, _L), jnp.float32),
            pltpu.VMEM((_BPW,), jnp.float32),
            pltpu.SemaphoreType.DMA,
        ],
    )
    out = run(xs_r, xd_r, emb_flat, dwb)
    return out.reshape(_B, 1)


# speedup vs baseline: 13.4719x; 9.3713x over previous
import jax
import jax.numpy as jnp
from jax import lax
from jax.experimental import pallas as pl
from jax.experimental.pallas import tpu as pltpu
from jax.experimental.pallas import tpu_sc as plsc

_B = 16384
_NS = 26
_ND = 13
_V = 1000000
_NC = 2
_NSUB = 16
_NW = _NC * _NSUB
_BPW = _B // _NW
_L = 16
_CHUNK = 128
_NCH = _NS * _BPW // _CHUNK
_QPF = _BPW // _CHUNK
_K = 8

_VP = 1000064
_W = _VP // 13
_FLAT = _NS * _VP


def _detile_body(in_ref, out_hbm, sem):
    f4 = pl.program_id(0)
    j = pl.program_id(1)
    for r in range(8):
        row = f4 * 8 + r

        @pl.when(row < _NS)
        def _():
            pltpu.make_async_copy(
                in_ref.at[r],
                out_hbm.at[pl.ds(row * _VP + j * _W, _W)],
                sem,
            ).start()
    for r in range(8):
        row = f4 * 8 + r

        @pl.when(row < _NS)
        def _():
            pltpu.make_async_copy(
                in_ref.at[r],
                out_hbm.at[pl.ds(row * _VP + j * _W, _W)],
                sem,
            ).wait()


def _gather_body(xs_hbm, xd_hbm, emb_hbm, dwb_hbm, out_hbm,
                 xs_v, idx_v, vals_v, xd_v, dw_v, acc_v, sem):
    w = lax.axis_index("s") * _NC + lax.axis_index("c")
    base = w * _BPW

    pltpu.sync_copy(xs_hbm.at[w], xs_v)
    pltpu.sync_copy(xd_hbm.at[w], xd_v)
    pltpu.sync_copy(dwb_hbm, dw_v)

    def build(j, carry):
        off = (j >> 2) * _VP
        for i in range(_CHUNK // _L):
            ids = xs_v[j, pl.ds(i * _L, _L)]
            idx_v[j, pl.ds(i * _L, _L)] = ids.astype(jnp.int32) + off
        return carry

    lax.fori_loop(0, _NCH, build, 0)

    def gather(g, carry):
        handles = []
        for b in range(_K):
            j = g * _K + b
            handles.append(
                pltpu.async_copy(emb_hbm.at[idx_v.at[j]], vals_v.at[j], sem))
        for h in handles:
            h.wait()
        return carry

    lax.fori_loop(0, _NCH // _K, gather, 0)

    def reduce(c, carry):
        q = c // (_CHUNK // _L)
        off = (c % (_CHUNK // _L)) * _L
        acc = vals_v[q, pl.ds(off, _L)]
        for f in range(1, _NS):
            acc = acc + vals_v[f * _QPF + q, pl.ds(off, _L)]
        for k in range(_ND):
            acc = acc + xd_v[k, pl.ds(c * _L, _L)] * dw_v[k]
        acc_v[pl.ds(c * _L, _L)] = acc
        return carry

    lax.fori_loop(0, _BPW // _L, reduce, 0)

    pltpu.sync_copy(acc_v, out_hbm.at[pl.ds(base, _BPW)])


@jax.jit
def kernel(X, emb_tables, dense_weight):
    xs_r = (X[:, :_NS].reshape(_NW, _QPF, _CHUNK, _NS)
            .transpose(0, 3, 1, 2).reshape(_NW, _NCH, _CHUNK))
    xd_r = jnp.concatenate(
        [X[:, _NS:], jnp.zeros((_B, 3), jnp.float32)], axis=1
    ).reshape(_NW, _BPW, _ND + 3).transpose(0, 2, 1)
    dwb = jnp.broadcast_to(dense_weight.reshape(_ND, 1), (_ND, _L))

    detile = pl.pallas_call(
        _detile_body,
        grid=(4, 13),
        in_specs=[pl.BlockSpec((8, _W), lambda f4, j: (f4, j))],
        out_specs=pl.BlockSpec(memory_space=pl.ANY),
        out_shape=jax.ShapeDtypeStruct((_FLAT,), jnp.float32),
        scratch_shapes=[pltpu.SemaphoreType.DMA],
    )
    emb_flat = detile(emb_tables)

    mesh = plsc.VectorSubcoreMesh(core_axis_name="c", subcore_axis_name="s")
    gather = pl.kernel(
        _gather_body,
        out_type=jax.ShapeDtypeStruct((_B,), jnp.float32),
        mesh=mesh,
        scratch_types=[
            pltpu.VMEM((_NCH, _CHUNK), jnp.float32),
            pltpu.VMEM((_NCH, _CHUNK), jnp.int32),
            pltpu.VMEM((_NCH, _CHUNK), jnp.float32),
            pltpu.VMEM((_ND + 3, _BPW), jnp.float32),
            pltpu.VMEM((_ND, _L), jnp.float32),
            pltpu.VMEM((_BPW,), jnp.float32),
            pltpu.SemaphoreType.DMA,
        ],
    )
    out = gather(xs_r, xd_r, emb_flat, dwb)
    return out.reshape(_B, 1)


# speedup vs baseline: 14.6716x; 1.0891x over previous
import jax
import jax.numpy as jnp
from jax import lax
from jax.experimental import pallas as pl
from jax.experimental.pallas import tpu as pltpu
from jax.experimental.pallas import tpu_sc as plsc

_B = 16384
_NS = 26
_ND = 13
_V = 1000000
_NC = 2
_NSUB = 16
_NW = _NC * _NSUB
_BPW = _B // _NW
_L = 16
_CHUNK = 128
_QPF = _BPW // _CHUNK
_K = 8

_VP = 1000064
_W = _VP // 13
_BANDS = ((0, 8), (8, 8), (16, 8), (24, 2))


def _make_detile(band, valid):
    def body(in_ref, out_hbm, sem):
        j = pl.program_id(0)
        copies = [
            pltpu.make_async_copy(
                in_ref.at[r],
                out_hbm.at[pl.ds(r * _VP + j * _W, _W)],
                sem,
            )
            for r in range(valid)
        ]
        for c in copies:
            c.start()
        for c in copies:
            c.wait()

    return pl.pallas_call(
        body,
        grid=(13,),
        in_specs=[pl.BlockSpec((8, _W), lambda j, b=band: (b, j))],
        out_specs=pl.BlockSpec(memory_space=pl.ANY),
        out_shape=jax.ShapeDtypeStruct((valid * _VP,), jnp.float32),
        scratch_shapes=[pltpu.SemaphoreType.DMA],
    )


def _make_gather(nf, with_dense):
    nch = nf * _QPF

    def body(xs_hbm, xd_hbm, emb_hbm, dwb_hbm, out_hbm,
             xs_v, idx_v, vals_v, xd_v, dw_v, acc_v, sem):
        w = lax.axis_index("s") * _NC + lax.axis_index("c")
        base = w * _BPW

        pltpu.sync_copy(xs_hbm.at[w], xs_v)
        if with_dense:
            pltpu.sync_copy(xd_hbm.at[w], xd_v)
            pltpu.sync_copy(dwb_hbm, dw_v)

        def build(j, carry):
            off = (j >> 2) * _VP
            for i in range(_CHUNK // _L):
                ids = xs_v[j, pl.ds(i * _L, _L)]
                idx_v[j, pl.ds(i * _L, _L)] = ids.astype(jnp.int32) + off
            return carry

        lax.fori_loop(0, nch, build, 0)

        def gather(g, carry):
            handles = []
            for b in range(_K):
                j = g * _K + b
                handles.append(
                    pltpu.async_copy(
                        emb_hbm.at[idx_v.at[j]], vals_v.at[j], sem))
            for h in handles:
                h.wait()
            return carry

        lax.fori_loop(0, nch // _K, gather, 0)

        def reduce(c, carry):
            q = c // (_CHUNK // _L)
            off = (c % (_CHUNK // _L)) * _L
            acc = vals_v[q, pl.ds(off, _L)]
            for f in range(1, nf):
                acc = acc + vals_v[f * _QPF + q, pl.ds(off, _L)]
            if with_dense:
                for k in range(_ND):
                    acc = acc + xd_v[k, pl.ds(c * _L, _L)] * dw_v[k]
            acc_v[pl.ds(c * _L, _L)] = acc
            return carry

        lax.fori_loop(0, _BPW // _L, reduce, 0)

        pltpu.sync_copy(acc_v, out_hbm.at[pl.ds(base, _BPW)])

    return pl.kernel(
        body,
        out_type=jax.ShapeDtypeStruct((_B,), jnp.float32),
        mesh=plsc.VectorSubcoreMesh(core_axis_name="c", subcore_axis_name="s"),
        scratch_types=[
            pltpu.VMEM((nch, _CHUNK), jnp.float32),
            pltpu.VMEM((nch, _CHUNK), jnp.int32),
            pltpu.VMEM((nch, _CHUNK), jnp.float32),
            pltpu.VMEM((_ND + 3, _BPW), jnp.float32),
            pltpu.VMEM((_ND, _L), jnp.float32),
            pltpu.VMEM((_BPW,), jnp.float32),
            pltpu.SemaphoreType.DMA,
        ],
    )


@jax.jit
def kernel(X, emb_tables, dense_weight):
    xs_r = (X[:, :_NS].reshape(_NW, _QPF, _CHUNK, _NS)
            .transpose(0, 3, 1, 2).reshape(_NW, _NS * _QPF, _CHUNK))
    xd_r = jnp.concatenate(
        [X[:, _NS:], jnp.zeros((_B, 3), jnp.float32)], axis=1
    ).reshape(_NW, _BPW, _ND + 3).transpose(0, 2, 1)
    dwb = jnp.broadcast_to(dense_weight.reshape(_ND, 1), (_ND, _L))

    partials = []
    for band, (f0, nf) in enumerate(_BANDS):
        emb_band = _make_detile(band, nf)(emb_tables)
        xs_g = xs_r[:, f0 * _QPF:(f0 + nf) * _QPF, :]
        run = _make_gather(nf, with_dense=(band == 0))
        partials.append(run(xs_g, xd_r, emb_band, dwb))
    out = partials[0] + partials[1] + partials[2] + partials[3]
    return out.reshape(_B, 1)
